# baseline (speedup 1.0000x reference)
import functools

import jax
import jax.numpy as jnp
from jax import lax
from jax.experimental import pallas as pl
from jax.experimental.pallas import tpu as pltpu
from jax.experimental.pallas import tpu_sc as plsc

N = 10000
NPAD = 10240
E = 160000
D_IN = 256
D_HID = 512
NC, NS = 2, 16
NW = NC * NS
EPT = E // NW
B = 125
NB = EPT // B
CW = 128
NCHUNK = D_HID // CW
STRIPE = NPAD // NS

def _deg_body(col3, ones_h, zer_h, out2, colv, onesv, acc, sem):
    c = lax.axis_index("c")
    s = lax.axis_index("s")
    wid = s * NC + c
    pltpu.sync_copy(col3.at[wid], colv)
    pltpu.sync_copy(ones_h, onesv)
    pltpu.sync_copy(zer_h, acc.at[pl.ds(s * STRIPE, STRIPE)])
    plsc.subcore_barrier()

    def body(j, carry):
        pltpu.sync_copy(onesv, acc.at[colv.at[j]], add=True)
        return carry

    lax.fori_loop(0, NB, body, 0)
    plsc.subcore_barrier()
    pltpu.sync_copy(acc.at[pl.ds(s * STRIPE, STRIPE)],
                    out2.at[c, pl.ds(s * STRIPE, STRIPE)])


@functools.cache
def _deg_call():
    return pl.kernel(
        _deg_body,
        out_type=jax.ShapeDtypeStruct((NC, NPAD), jnp.float32),
        mesh=plsc.VectorSubcoreMesh(core_axis_name="c", subcore_axis_name="s"),
        scratch_types=[
            pltpu.VMEM((NB, B), jnp.int32),
            pltpu.VMEM((B,), jnp.float32),
            pltpu.VMEM_SHARED((NPAD,), jnp.float32),
            pltpu.SemaphoreType.DMA,
        ],
    )


def _scat_body(nch, y4, row3, col3, zer2_h, out4,
               rowv, colv, buf0, buf1, acc, sem0, sem1):
    c = lax.axis_index("c")
    s = lax.axis_index("s")
    wid = s * NC + c
    pltpu.sync_copy(row3.at[wid], rowv)
    pltpu.sync_copy(col3.at[wid], colv)

    for k in range(nch):
        yk = y4.at[k]
        pltpu.sync_copy(zer2_h, acc.at[pl.ds(s * STRIPE, STRIPE)])
        plsc.subcore_barrier()

        pltpu.async_copy(yk.at[rowv.at[0]], buf0, sem0)

        def body(j2, carry):
            j = 2 * j2
            pltpu.make_async_copy(yk.at[rowv.at[j]], buf0, sem0).wait()
            pltpu.async_copy(yk.at[rowv.at[j + 1]], buf1, sem1)
            pltpu.sync_copy(buf0, acc.at[colv.at[j]], add=True)
            pltpu.make_async_copy(yk.at[rowv.at[j + 1]], buf1, sem1).wait()

            @pl.when(j2 + 1 < NB // 2)
            def _():
                pltpu.async_copy(yk.at[rowv.at[j + 2]], buf0, sem0)

            pltpu.sync_copy(buf1, acc.at[colv.at[j + 1]], add=True)
            return carry

        lax.fori_loop(0, NB // 2, body, 0)
        plsc.subcore_barrier()
        pltpu.sync_copy(acc.at[pl.ds(s * STRIPE, STRIPE)],
                        out4.at[c, k, pl.ds(s * STRIPE, STRIPE)])


@functools.cache
def _scat_call(nch):
    return pl.kernel(
        functools.partial(_scat_body, nch),
        out_type=jax.ShapeDtypeStruct((NC, nch, NPAD, CW), jnp.float32),
        mesh=plsc.VectorSubcoreMesh(core_axis_name="c", subcore_axis_name="s"),
        scratch_types=[
            pltpu.VMEM((NB, B), jnp.int32),
            pltpu.VMEM((NB, B), jnp.int32),
            pltpu.VMEM((B, CW), jnp.float32),
            pltpu.VMEM((B, CW), jnp.float32),
            pltpu.VMEM_SHARED((NPAD, CW), jnp.float32),
            pltpu.SemaphoreType.DMA,
            pltpu.SemaphoreType.DMA,
        ],
        cost_estimate=pl.CostEstimate(
            flops=nch * E * CW,
            transcendentals=0,
            bytes_accessed=2 * nch * (E * CW * 4 + NC * NPAD * CW * 4),
        ),
    )


_MT = 2000


def _d_from(degT_blk):
    deg = degT_blk[:, 0:1] + degT_blk[:, 1:2] + 1.0
    return lax.rsqrt(deg)


def _mm1_body(degT_ref, x_ref, w1_ref, out_ref):
    d = _d_from(degT_ref[...])
    xt = lax.dot_general(x_ref[...].astype(jnp.bfloat16), w1_ref[...],
                         (((1,), (1,)), ((), ())),
                         preferred_element_type=jnp.float32)
    out_ref[0] = xt * d


def _mm1(degT, x, w1half):
    return pl.pallas_call(
        _mm1_body,
        grid=(N // _MT, 2),
        in_specs=[
            pl.BlockSpec((_MT, 2), lambda m, n: (m, 0)),
            pl.BlockSpec((_MT, D_IN), lambda m, n: (m, 0)),
            pl.BlockSpec((CW, D_IN), lambda m, n: (n, 0)),
        ],
        out_specs=pl.BlockSpec((1, _MT, CW), lambda m, n: (n, m, 0)),
        out_shape=jax.ShapeDtypeStruct((2, N, CW), jnp.float32),
    )(degT, x, w1half)


def _h_chunks(d, s_ref, y_ref, b_ref, w_ref):
    acc = jnp.zeros((_MT, CW), jnp.float32)
    for k in range(2):
        hk = d * (s_ref[0, k] + s_ref[1, k] + y_ref[k]) + b_ref[k][None, :]
        hk = jnp.maximum(hk, 0.0)
        acc = acc + lax.dot_general(hk.astype(jnp.bfloat16),
                                    w_ref[0, k * CW:(k + 1) * CW, :],
                                    (((1,), (0,)), ((), ())),
                                    preferred_element_type=jnp.float32)
    return acc


def _mm2a_body(degT_ref, s1_ref, y1_ref, b1_ref, w2t_ref, out_ref):
    d = _d_from(degT_ref[...])
    out_ref[0] = _h_chunks(d, s1_ref, y1_ref, b1_ref, w2t_ref)


def _mm2a(degT, s1a, y1a, b1a, w2ta):
    return pl.pallas_call(
        _mm2a_body,
        grid=(N // _MT, NCHUNK),
        in_specs=[
            pl.BlockSpec((_MT, 2), lambda m, n: (m, 0)),
            pl.BlockSpec((NC, 2, _MT, CW), lambda m, n: (0, 0, m, 0)),
            pl.BlockSpec((2, _MT, CW), lambda m, n: (0, m, 0)),
            pl.BlockSpec((2, CW), lambda m, n: (0, 0)),
            pl.BlockSpec((1, 2 * CW, CW), lambda m, n: (n, 0, 0)),
        ],
        out_specs=pl.BlockSpec((1, _MT, CW), lambda m, n: (n, m, 0)),
        out_shape=jax.ShapeDtypeStruct((NCHUNK, N, CW), jnp.float32),
    )(degT, s1a, y1a, b1a, w2ta)


def _mm2b_body(degT_ref, s1_ref, y1_ref, b1_ref, w2t_ref, y2p_ref, out_ref):
    d = _d_from(degT_ref[...])
    acc = y2p_ref[0] + _h_chunks(d, s1_ref, y1_ref, b1_ref, w2t_ref)
    out_ref[0] = acc * d


def _mm2b(n0, degT, s1b, y1b, b1b, w2tb, y2p):
    return pl.pallas_call(
        _mm2b_body,
        grid=(N // _MT, 2),
        in_specs=[
            pl.BlockSpec((_MT, 2), lambda m, n: (m, 0)),
            pl.BlockSpec((NC, 2, _MT, CW), lambda m, n: (0, 0, m, 0)),
            pl.BlockSpec((2, _MT, CW), lambda m, n: (0, m, 0)),
            pl.BlockSpec((2, CW), lambda m, n: (0, 0)),
            pl.BlockSpec((1, 2 * CW, CW), lambda m, n: (n, 0, 0)),
            pl.BlockSpec((1, _MT, CW), lambda m, n, n0=n0: (n + n0, m, 0)),
        ],
        out_specs=pl.BlockSpec((1, _MT, CW), lambda m, n: (n, m, 0)),
        out_shape=jax.ShapeDtypeStruct((2, N, CW), jnp.float32),
    )(degT, s1b, y1b, b1b, w2tb, y2p)


def _ep3_body(degT_ref, s2_ref, y2_ref, b2_ref, out_ref):
    d = _d_from(degT_ref[...])
    b = b2_ref[pl.program_id(1)][None, :]
    out_ref[...] = d * (s2_ref[0, 0] + s2_ref[1, 0] + y2_ref[0]) + b


def _ep3h(degT, s2h, y2h, b2h):
    return pl.pallas_call(
        _ep3_body,
        grid=(N // _MT, 2),
        in_specs=[
            pl.BlockSpec((_MT, 2), lambda m, n: (m, 0)),
            pl.BlockSpec((NC, 1, _MT, CW), lambda m, n: (0, n, m, 0)),
            pl.BlockSpec((1, _MT, CW), lambda m, n: (n, m, 0)),
            pl.BlockSpec((2, CW), lambda m, n: (0, 0)),
        ],
        out_specs=pl.BlockSpec((_MT, CW), lambda m, n: (m, n)),
        out_shape=jax.ShapeDtypeStruct((N, 2 * CW), jnp.float32),
    )(degT, s2h, y2h, b2h)


@jax.jit
def kernel(x, edge_index, W1, b1, W2, b2):
    ei = edge_index.astype(jnp.int32)
    row3 = ei[0].reshape(NW, NB, B)
    col3 = ei[1].reshape(NW, NB, B)
    ones_h = jnp.ones((B,), jnp.float32)
    zer1 = jnp.zeros((STRIPE,), jnp.float32)
    zer2 = jnp.zeros((STRIPE, CW), jnp.float32)
    b1r = b1.reshape(NCHUNK, CW)
    b2r = b2.reshape(NCHUNK, CW)
    w1h = W1.astype(jnp.bfloat16)
    w2t = W2.T.reshape(D_HID, NCHUNK, CW).transpose(1, 0, 2).astype(jnp.bfloat16)

    scat = _scat_call(2)
    deg2 = _deg_call()(col3, ones_h, zer1)
    degT = deg2.T

    y1a = _mm1(degT, x, w1h[:2 * CW])
    y1b = _mm1(degT, x, w1h[2 * CW:])
    s1a = scat(y1a, row3, col3, zer2)
    s1b = scat(y1b, row3, col3, zer2)
    y2p = _mm2a(degT, s1a, y1a, b1r[:2], w2t[:, :2 * CW, :])
    y2a = _mm2b(0, degT, s1b, y1b, b1r[2:], w2t[:2, 2 * CW:, :], y2p)
    y2b = _mm2b(2, degT, s1b, y1b, b1r[2:], w2t[2:, 2 * CW:, :], y2p)
    s2a = scat(y2a, row3, col3, zer2)
    s2b = scat(y2b, row3, col3, zer2)
    outa = _ep3h(degT, s2a, y2a, b2r[:2])
    outb = _ep3h(degT, s2b, y2b, b2r[2:])
    return jnp.concatenate([outa, outb], axis=1)


# speedup vs baseline: 1.0131x; 1.0131x over previous
import functools

import jax
import jax.numpy as jnp
from jax import lax
from jax.experimental import pallas as pl
from jax.experimental.pallas import tpu as pltpu
from jax.experimental.pallas import tpu_sc as plsc

N = 10000
NPAD = 10240
E = 160000
D_IN = 256
D_HID = 512
NC, NS = 2, 16
NW = NC * NS
EPT = E // NW
B = 125
NB = EPT // B
CW = 128
NCHUNK = D_HID // CW
STRIPE = NPAD // NS

def _deg_body(col3, ones_h, zer_h, out2, colv, onesv, acc, sem):
    c = lax.axis_index("c")
    s = lax.axis_index("s")
    wid = s * NC + c
    pltpu.sync_copy(col3.at[wid], colv)
    pltpu.sync_copy(ones_h, onesv)
    pltpu.sync_copy(zer_h, acc.at[pl.ds(s * STRIPE, STRIPE)])
    plsc.subcore_barrier()

    def body(j, carry):
        pltpu.sync_copy(onesv, acc.at[colv.at[j]], add=True)
        return carry

    lax.fori_loop(0, NB, body, 0)
    plsc.subcore_barrier()
    pltpu.sync_copy(acc.at[pl.ds(s * STRIPE, STRIPE)],
                    out2.at[c, pl.ds(s * STRIPE, STRIPE)])


@functools.cache
def _deg_call():
    return pl.kernel(
        _deg_body,
        out_type=jax.ShapeDtypeStruct((NC, NPAD), jnp.float32),
        mesh=plsc.VectorSubcoreMesh(core_axis_name="c", subcore_axis_name="s"),
        scratch_types=[
            pltpu.VMEM((NB, B), jnp.int32),
            pltpu.VMEM((B,), jnp.float32),
            pltpu.VMEM_SHARED((NPAD,), jnp.float32),
            pltpu.SemaphoreType.DMA,
        ],
    )


def _scat_body(nch, y4, row3, col3, zer2_h, out4,
               rowv, colv, buf0, buf1, acc, sem0, sem1):
    c = lax.axis_index("c")
    s = lax.axis_index("s")
    wid = s * NC + c
    pltpu.sync_copy(row3.at[wid], rowv)
    pltpu.sync_copy(col3.at[wid], colv)

    for k in range(nch):
        yk = y4.at[k]
        pltpu.sync_copy(zer2_h, acc.at[pl.ds(s * STRIPE, STRIPE)])
        plsc.subcore_barrier()
        pltpu.async_copy(yk.at[rowv.at[0]], buf0, sem0)

        def body(j2, carry):
            j = 2 * j2
            pltpu.make_async_copy(yk.at[rowv.at[j]], buf0, sem0).wait()
            pltpu.async_copy(yk.at[rowv.at[j + 1]], buf1, sem1)
            pltpu.sync_copy(buf0, acc.at[colv.at[j]], add=True)
            pltpu.make_async_copy(yk.at[rowv.at[j + 1]], buf1, sem1).wait()

            @pl.when(j2 + 1 < NB // 2)
            def _():
                pltpu.async_copy(yk.at[rowv.at[j + 2]], buf0, sem0)

            pltpu.sync_copy(buf1, acc.at[colv.at[j + 1]], add=True)
            return carry

        lax.fori_loop(0, NB // 2, body, 0)
        plsc.subcore_barrier()
        pltpu.sync_copy(acc.at[pl.ds(s * STRIPE, STRIPE)],
                        out4.at[c, k, pl.ds(s * STRIPE, STRIPE)])


@functools.cache
def _scat_call(nch):
    return pl.kernel(
        functools.partial(_scat_body, nch),
        out_type=jax.ShapeDtypeStruct((NC, nch, NPAD, CW), jnp.float32),
        mesh=plsc.VectorSubcoreMesh(core_axis_name="c", subcore_axis_name="s"),
        scratch_types=[
            pltpu.VMEM((NB, B), jnp.int32),
            pltpu.VMEM((NB, B), jnp.int32),
            pltpu.VMEM((B, CW), jnp.float32),
            pltpu.VMEM((B, CW), jnp.float32),
            pltpu.VMEM_SHARED((NPAD, CW), jnp.float32),
            pltpu.SemaphoreType.DMA,
            pltpu.SemaphoreType.DMA,
        ],
    )


_MT = 2000


def _d_from(degT_blk):
    deg = degT_blk[:, 0:1] + degT_blk[:, 1:2] + 1.0
    return lax.rsqrt(deg)


def _mm1_body(degT_ref, x_ref, w1_ref, out_ref):
    d = _d_from(degT_ref[...])
    xt = lax.dot_general(x_ref[...].astype(jnp.bfloat16), w1_ref[...],
                         (((1,), (1,)), ((), ())),
                         preferred_element_type=jnp.float32)
    out_ref[0] = xt * d


def _mm1(degT, x, w1):
    return pl.pallas_call(
        _mm1_body,
        grid=(N // _MT, NCHUNK),
        in_specs=[
            pl.BlockSpec((_MT, 2), lambda m, n: (m, 0)),
            pl.BlockSpec((_MT, D_IN), lambda m, n: (m, 0)),
            pl.BlockSpec((CW, D_IN), lambda m, n: (n, 0)),
        ],
        out_specs=pl.BlockSpec((1, _MT, CW), lambda m, n: (n, m, 0)),
        out_shape=jax.ShapeDtypeStruct((NCHUNK, N, CW), jnp.float32),
    )(degT, x, w1)


def _mm2_body(degT_ref, s1_ref, y1_ref, b1_ref, w2t_ref, out_ref):
    d = _d_from(degT_ref[...])
    acc = jnp.zeros((_MT, CW), jnp.float32)
    for k in range(NCHUNK):
        hk = d * (s1_ref[0, k] + s1_ref[1, k] + y1_ref[k]) + b1_ref[k][None, :]
        hk = jnp.maximum(hk, 0.0)
        acc = acc + lax.dot_general(hk.astype(jnp.bfloat16),
                                    w2t_ref[0, k * CW:(k + 1) * CW, :],
                                    (((1,), (0,)), ((), ())),
                                    preferred_element_type=jnp.float32)
    out_ref[0] = acc * d


def _mm2(degT, s1, y1, b1r, w2t):
    return pl.pallas_call(
        _mm2_body,
        grid=(N // _MT, NCHUNK),
        in_specs=[
            pl.BlockSpec((_MT, 2), lambda m, n: (m, 0)),
            pl.BlockSpec((NC, NCHUNK, _MT, CW), lambda m, n: (0, 0, m, 0)),
            pl.BlockSpec((NCHUNK, _MT, CW), lambda m, n: (0, m, 0)),
            pl.BlockSpec((NCHUNK, CW), lambda m, n: (0, 0)),
            pl.BlockSpec((1, D_HID, CW), lambda m, n: (n, 0, 0)),
        ],
        out_specs=pl.BlockSpec((1, _MT, CW), lambda m, n: (n, m, 0)),
        out_shape=jax.ShapeDtypeStruct((NCHUNK, N, CW), jnp.float32),
    )(degT, s1, y1, b1r, w2t)


def _ep3_body(degT_ref, s2_ref, y2_ref, b2_ref, out_ref):
    d = _d_from(degT_ref[...])
    b = b2_ref[pl.program_id(1)][None, :]
    out_ref[...] = d * (s2_ref[0, 0] + s2_ref[1, 0] + y2_ref[0]) + b


def _ep3(degT, s2, y2, b2r):
    return pl.pallas_call(
        _ep3_body,
        grid=(N // _MT, NCHUNK),
        in_specs=[
            pl.BlockSpec((_MT, 2), lambda m, n: (m, 0)),
            pl.BlockSpec((NC, 1, _MT, CW), lambda m, n: (0, n, m, 0)),
            pl.BlockSpec((1, _MT, CW), lambda m, n: (n, m, 0)),
            pl.BlockSpec((NCHUNK, CW), lambda m, n: (0, 0)),
        ],
        out_specs=pl.BlockSpec((_MT, CW), lambda m, n: (m, n)),
        out_shape=jax.ShapeDtypeStruct((N, D_HID), jnp.float32),
    )(degT, s2, y2, b2r)


@jax.jit
def kernel(x, edge_index, W1, b1, W2, b2):
    ei = edge_index.astype(jnp.int32)
    row3 = ei[0].reshape(NW, NB, B)
    col3 = ei[1].reshape(NW, NB, B)
    ones_h = jnp.ones((B,), jnp.float32)
    zer1 = jnp.zeros((STRIPE,), jnp.float32)
    zer2 = jnp.zeros((STRIPE, CW), jnp.float32)
    b1r = b1.reshape(NCHUNK, CW)
    b2r = b2.reshape(NCHUNK, CW)
    w1h = W1.astype(jnp.bfloat16)
    w2t = W2.T.reshape(D_HID, NCHUNK, CW).transpose(1, 0, 2).astype(jnp.bfloat16)

    scat = _scat_call(NCHUNK)
    deg2 = _deg_call()(col3, ones_h, zer1)
    degT = deg2.T

    y1 = _mm1(degT, x, w1h)
    s1 = scat(y1, row3, col3, zer2)
    y2 = _mm2(degT, s1, y1, b1r, w2t)
    s2 = scat(y2, row3, col3, zer2)
    return _ep3(degT, s2, y2, b2r)
